# baseline (speedup 1.0000x reference)
import functools

import jax
import jax.numpy as jnp
from jax import lax
from jax.experimental import pallas as pl
from jax.experimental.pallas import tpu as pltpu
from jax.experimental.pallas import tpu_sc as plsc

NUM_GRAPHS = 64
REUSE_WEIGHT = 0.01

NC = 2
NS = 16
L = 16
NW = NC * NS
ROWS = 100000
D = 128
RPW = ROWS // NW
CHUNK = 125
CHUNK_PAD = 128
NCHUNK = RPW // CHUNK
SEG_PAD = NUM_GRAPHS + 1
NBUF = 4


def _seg_body(codes_hbm, batch_hbm, sums_out, cnts_out,
              idx_v, bufs, ones_v, sums_sh, cnts_sh,
              load_sems, scat_sems, cnt_sem):
    c = lax.axis_index("c")
    s = lax.axis_index("s")
    wid = s * NC + c
    base = wid * RPW

    zvec = jnp.zeros((L,), jnp.float32)

    @pl.when(s == 0)
    def _init():
        def zrow(i, carry):
            for jj in range(D // L):
                bufs[0, i, pl.ds(jj * L, L)] = zvec
            ones_v[i, :] = zvec
            return carry
        lax.fori_loop(0, SEG_PAD, zrow, 0)
        pltpu.sync_copy(bufs.at[0].at[pl.ds(0, SEG_PAD)], sums_sh)
        pltpu.sync_copy(ones_v.at[pl.ds(0, SEG_PAD)], cnts_sh)

    plsc.subcore_barrier()

    ovec = jnp.ones((L,), jnp.float32)

    def orow(i, carry):
        ones_v[i, :] = ovec
        return carry
    lax.fori_loop(0, CHUNK_PAD, orow, 0)

    def load(j):
        pltpu.async_copy(codes_hbm.at[pl.ds(base + j * CHUNK, CHUNK)],
                         bufs.at[j % NBUF].at[pl.ds(0, CHUNK)],
                         load_sems.at[j % NBUF])

    def wait_load(j):
        pltpu.make_async_copy(codes_hbm.at[pl.ds(base + j * CHUNK, CHUNK)],
                              bufs.at[j % NBUF].at[pl.ds(0, CHUNK)],
                              load_sems.at[j % NBUF]).wait()

    def scatter(j):
        pltpu.async_copy(bufs.at[j % NBUF], sums_sh.at[idx_v.at[j]],
                         scat_sems.at[j % 2], add=True)

    def wait_scatter(j):
        pltpu.make_async_copy(bufs.at[j % NBUF], sums_sh.at[idx_v.at[j]],
                              scat_sems.at[j % 2]).wait()

    load(0)
    load(1)
    pltpu.sync_copy(batch_hbm.at[pl.ds(wid * NCHUNK, NCHUNK)], idx_v)

    for j in range(NCHUNK):
        if j + 2 < NCHUNK:
            if j >= 2:
                wait_scatter(j - 2)
            load(j + 2)
        wait_load(j)
        scatter(j)
        pltpu.async_copy(ones_v, cnts_sh.at[idx_v.at[j]], cnt_sem, add=True)

    for j in range(NCHUNK - 4, NCHUNK):
        wait_scatter(j)
    for j in range(NCHUNK):
        pltpu.make_async_copy(ones_v, cnts_sh.at[idx_v.at[0]], cnt_sem).wait()

    plsc.subcore_barrier()

    @pl.when(s == 0)
    def _flush():
        pltpu.sync_copy(sums_sh, bufs.at[0].at[pl.ds(0, SEG_PAD)])
        pltpu.sync_copy(bufs.at[0].at[pl.ds(0, SEG_PAD)], sums_out.at[c])
        pltpu.sync_copy(cnts_sh, ones_v.at[pl.ds(0, SEG_PAD)])
        pltpu.sync_copy(ones_v.at[pl.ds(0, SEG_PAD)], cnts_out.at[c])


@functools.lru_cache(maxsize=1)
def _make_seg_reduce():
    return functools.partial(
        pl.kernel,
        out_type=[
            jax.ShapeDtypeStruct((NC, SEG_PAD, D), jnp.float32),
            jax.ShapeDtypeStruct((NC, SEG_PAD, L), jnp.float32),
        ],
        mesh=plsc.VectorSubcoreMesh(core_axis_name="c", subcore_axis_name="s"),
        scratch_types=[
            pltpu.VMEM((NCHUNK, CHUNK_PAD), jnp.int32),
            pltpu.VMEM((NBUF, CHUNK_PAD, D), jnp.float32),
            pltpu.VMEM((CHUNK_PAD, L), jnp.float32),
            pltpu.VMEM_SHARED((SEG_PAD, D), jnp.float32),
            pltpu.VMEM_SHARED((SEG_PAD, L), jnp.float32),
            pltpu.SemaphoreType.DMA((NBUF,)),
            pltpu.SemaphoreType.DMA((2,)),
            pltpu.SemaphoreType.DMA,
        ],
        compiler_params=pltpu.CompilerParams(use_tc_tiling_on_sc=False),
    )(_seg_body)


def _fin_body(s_ref, c_ref, o_ref):
    sums = s_ref[0, :NUM_GRAPHS, :] + s_ref[1, :NUM_GRAPHS, :]
    counts = c_ref[0, :NUM_GRAPHS, 0:1] + c_ref[1, :NUM_GRAPHS, 0:1]
    means = sums / counts
    mu = jnp.mean(means, axis=0, keepdims=True)
    dev = means - mu
    var = jnp.sum(dev * dev, axis=0) / (NUM_GRAPHS - 1)
    o_ref[...] = jnp.reshape(-REUSE_WEIGHT * jnp.mean(var), (1, 1))


def kernel(sparse_codes, batch):
    batch2d = jnp.pad(
        batch.astype(jnp.int32).reshape(NW * NCHUNK, CHUNK),
        ((0, 0), (0, CHUNK_PAD - CHUNK)),
        constant_values=NUM_GRAPHS,
    )
    sums, cnts = _make_seg_reduce()(sparse_codes, batch2d)
    out = pl.pallas_call(
        _fin_body,
        out_shape=jax.ShapeDtypeStruct((1, 1), jnp.float32),
    )(sums, cnts)
    return out[0, 0]


# speedup vs baseline: 1.6152x; 1.6152x over previous
import functools

import jax
import jax.numpy as jnp
from jax import lax
from jax.experimental import pallas as pl
from jax.experimental.pallas import tpu as pltpu
from jax.experimental.pallas import tpu_sc as plsc

NUM_GRAPHS = 64
REUSE_WEIGHT = 0.01

NC = 2
NS = 16
L = 16
NW = NC * NS
ROWS = 100000
D = 128
RPW = ROWS // NW
CHUNK = 125
CHUNK_PAD = 128
NCHUNK = RPW // CHUNK
NCHUNK_PAD = 32
SEG_PAD = NUM_GRAPHS + 1
NBUF = 4
RUNROLL = 5


def _seg_body(codes_hbm, batch_hbm, fid_hbm, sums_out, cnts_out,
              idx_v, bufs, ones_v, csum_v, c125_v, fid_v,
              sums_sh, cnts_sh, load_sems):
    c = lax.axis_index("c")
    s = lax.axis_index("s")
    wid = s * NC + c
    base = wid * RPW

    zvec = jnp.zeros((L,), jnp.float32)

    @pl.when(s == 0)
    def _init():
        def zrow(i, carry):
            for jj in range(D // L):
                bufs[0, i, pl.ds(jj * L, L)] = zvec
            ones_v[i, :] = zvec
            return carry
        lax.fori_loop(0, SEG_PAD, zrow, 0)
        pltpu.sync_copy(bufs.at[0].at[pl.ds(0, SEG_PAD)], sums_sh)
        pltpu.sync_copy(ones_v.at[pl.ds(0, SEG_PAD)], cnts_sh)

    plsc.subcore_barrier()

    ovec = jnp.ones((L,), jnp.float32)

    def orow(i, carry):
        ones_v[i, :] = ovec
        return carry
    lax.fori_loop(0, CHUNK_PAD, orow, 0)

    cvec = jnp.full((L,), float(CHUNK), jnp.float32)
    for i in range(NCHUNK_PAD):
        c125_v[i, :] = cvec

    def load(j):
        pltpu.async_copy(codes_hbm.at[pl.ds(base + j * CHUNK, CHUNK)],
                         bufs.at[j % NBUF].at[pl.ds(0, CHUNK)],
                         load_sems.at[j % NBUF])

    def wait_load(j):
        pltpu.make_async_copy(codes_hbm.at[pl.ds(base + j * CHUNK, CHUNK)],
                              bufs.at[j % NBUF].at[pl.ds(0, CHUNK)],
                              load_sems.at[j % NBUF]).wait()

    load(0)
    load(1)
    pltpu.sync_copy(batch_hbm.at[pl.ds(wid * NCHUNK, NCHUNK)], idx_v)
    pltpu.sync_copy(fid_hbm.at[wid], fid_v)

    lane_iota = lax.iota(jnp.int32, L)

    for j in range(NCHUNK):
        if j + 2 < NCHUNK:
            load(j + 2)
        wait_load(j)
        buf = bufs.at[j % NBUF]
        fslice = fid_v[pl.ds((j // L) * L, L)]
        fsel = jnp.where(lane_iota == (j % L), fslice, 0)
        fid_j = lax.reduce_max(fsel, axes=(0,))
        mixed = fid_j == NUM_GRAPHS

        @pl.when(mixed)
        def _fallback():
            pltpu.sync_copy(buf, sums_sh.at[idx_v.at[j]], add=True)
            pltpu.sync_copy(ones_v, cnts_sh.at[idx_v.at[j]], add=True)

        @pl.when(jnp.logical_not(mixed))
        def _reduce():
            def rbody(r5, accs):
                accs = list(accs)
                for rr in range(RUNROLL):
                    r = r5 * RUNROLL + rr
                    for jj in range(D // L):
                        accs[jj] = accs[jj] + buf[r, pl.ds(jj * L, L)]
                return tuple(accs)
            accs = lax.fori_loop(0, CHUNK // RUNROLL, rbody,
                                 tuple(zvec for _ in range(D // L)))
            for jj in range(D // L):
                csum_v[j, pl.ds(jj * L, L)] = accs[jj]

    pltpu.sync_copy(csum_v, sums_sh.at[fid_v], add=True)
    pltpu.sync_copy(c125_v, cnts_sh.at[fid_v], add=True)

    plsc.subcore_barrier()

    @pl.when(s == 0)
    def _flush():
        pltpu.sync_copy(sums_sh, bufs.at[0].at[pl.ds(0, SEG_PAD)])
        pltpu.sync_copy(bufs.at[0].at[pl.ds(0, SEG_PAD)], sums_out.at[c])
        pltpu.sync_copy(cnts_sh, ones_v.at[pl.ds(0, SEG_PAD)])
        pltpu.sync_copy(ones_v.at[pl.ds(0, SEG_PAD)], cnts_out.at[c])


@functools.lru_cache(maxsize=1)
def _make_seg_reduce():
    return functools.partial(
        pl.kernel,
        out_type=[
            jax.ShapeDtypeStruct((NC, SEG_PAD, D), jnp.float32),
            jax.ShapeDtypeStruct((NC, SEG_PAD, L), jnp.float32),
        ],
        mesh=plsc.VectorSubcoreMesh(core_axis_name="c", subcore_axis_name="s"),
        scratch_types=[
            pltpu.VMEM((NCHUNK, CHUNK_PAD), jnp.int32),
            pltpu.VMEM((NBUF, CHUNK_PAD, D), jnp.float32),
            pltpu.VMEM((CHUNK_PAD, L), jnp.float32),
            pltpu.VMEM((NCHUNK_PAD, D), jnp.float32),
            pltpu.VMEM((NCHUNK_PAD, L), jnp.float32),
            pltpu.VMEM((NCHUNK_PAD,), jnp.int32),
            pltpu.VMEM_SHARED((SEG_PAD, D), jnp.float32),
            pltpu.VMEM_SHARED((SEG_PAD, L), jnp.float32),
            pltpu.SemaphoreType.DMA((NBUF,)),
        ],
        compiler_params=pltpu.CompilerParams(use_tc_tiling_on_sc=False,
                                             needs_layout_passes=False),
    )(_seg_body)


def _fin_body(s_ref, c_ref, o_ref):
    sums = s_ref[0, :NUM_GRAPHS, :] + s_ref[1, :NUM_GRAPHS, :]
    counts = c_ref[0, :NUM_GRAPHS, 0:1] + c_ref[1, :NUM_GRAPHS, 0:1]
    means = sums / counts
    mu = jnp.mean(means, axis=0, keepdims=True)
    dev = means - mu
    var = jnp.sum(dev * dev, axis=0) / (NUM_GRAPHS - 1)
    o_ref[...] = jnp.reshape(-REUSE_WEIGHT * jnp.mean(var), (1, 1))


def kernel(sparse_codes, batch):
    batch_i = batch.astype(jnp.int32)
    batch2d = jnp.pad(
        batch_i.reshape(NW * NCHUNK, CHUNK),
        ((0, 0), (0, CHUNK_PAD - CHUNK)),
        constant_values=NUM_GRAPHS,
    )
    firsts = batch_i[0::CHUNK]
    lasts = batch_i[CHUNK - 1::CHUNK]
    fid = jnp.where(firsts == lasts, firsts, NUM_GRAPHS)
    fid2d = jnp.pad(
        fid.reshape(NW, NCHUNK),
        ((0, 0), (0, NCHUNK_PAD - NCHUNK)),
        constant_values=NUM_GRAPHS,
    )
    sums, cnts = _make_seg_reduce()(sparse_codes, batch2d, fid2d)
    out = pl.pallas_call(
        _fin_body,
        out_shape=jax.ShapeDtypeStruct((1, 1), jnp.float32),
    )(sums, cnts)
    return out[0, 0]
